# baseline (speedup 1.0000x reference)
import jax
import jax.numpy as jnp
from jax.experimental import pallas as pl
from jax.experimental.pallas import tpu as pltpu


def _body(s_ref, l0_ref, l1_ref, l2_ref, l3_ref, o_ref, e_ref, out_ref,
          acc_ref):
    i = pl.program_id(0)
    j = pl.program_id(1)
    gi = pl.num_programs(0)
    gj = pl.num_programs(1)

    @pl.when((i == 0) & (j == 0))
    def _():
        acc_ref[0] = 0.0
        acc_ref[1] = 0.0
        acc_ref[2] = 0.0

    R = l0_ref.shape[1]
    C = s_ref.shape[2]

    lab3 = jnp.concatenate(
        [lr[...].reshape(2 * R, 1) for lr in (l0_ref, l1_ref, l2_ref,
                                              l3_ref)],
        axis=0,
    )
    fg3 = (lab3 != 0).astype(jnp.float32)
    nfg_part = jnp.sum(fg3)

    s3 = s_ref[...].reshape(8 * R, C)
    ones_v = jnp.ones((C, 128), dtype=jnp.float32)
    ex = jnp.exp(s3)
    sum_ex = jax.lax.dot_general(
        ex, ones_v, (((1,), (0,)), ((), ())),
        preferred_element_type=jnp.float32,
    )[:, 0:1]
    lse_sum = jnp.sum(jnp.log(sum_ex))

    iota3 = jax.lax.broadcasted_iota(jnp.int32, (8 * R, C), 1)
    oh = jnp.where(iota3 == lab3, s3, 0.0)
    picked = jax.lax.dot_general(
        oh, ones_v, (((1,), (0,)), ((), ())),
        preferred_element_type=jnp.float32,
    )[:, 0:1]
    picked_sum = jnp.sum(picked)

    labL = jnp.swapaxes(lab3, 0, 1)
    lab_lane = jnp.concatenate(
        [labL[:, b * R : (b + 1) * R] for b in range(8)], axis=0
    )
    fgT = (jnp.swapaxes(lab_lane, 0, 1) != 0).astype(jnp.float32)

    d = o_ref[...] - e_ref[...]
    dT = jnp.swapaxes(d, 0, 1)
    dT3 = dT.reshape(R, 4, 8)
    ad = jnp.abs(dT3)
    sl1 = jnp.where(ad < 1.0, 0.5 * dT3 * dT3, ad - 0.5)
    sl1_part = jnp.sum(sl1 * fgT[:, None, :])

    acc_ref[0] += lse_sum - picked_sum
    acc_ref[1] += nfg_part
    acc_ref[2] += sl1_part

    @pl.when((i == gi - 1) & (j == gj - 1))
    def _():
        nf = acc_ref[1]
        cls = acc_ref[0] / nf
        reg = acc_ref[2] / nf
        out_ref[0] = cls
        out_ref[1] = reg
        out_ref[2] = cls + reg


def kernel(offsets, scores, assigned_labels, encoded_bboxes):
    B, A, C = scores.shape
    R = 1024
    GB = B // 8
    GA = A // R

    off2 = offsets.reshape(B, A * 4)
    enc2 = encoded_bboxes.reshape(B, A * 4)

    out = pl.pallas_call(
        _body,
        grid=(GB, GA),
        in_specs=[
            pl.BlockSpec((8, R, C), lambda i, j: (i, j, 0)),
            pl.BlockSpec((2, R, 1), lambda i, j: (4 * i, j, 0)),
            pl.BlockSpec((2, R, 1), lambda i, j: (4 * i + 1, j, 0)),
            pl.BlockSpec((2, R, 1), lambda i, j: (4 * i + 2, j, 0)),
            pl.BlockSpec((2, R, 1), lambda i, j: (4 * i + 3, j, 0)),
            pl.BlockSpec((8, 4 * R), lambda i, j: (i, j)),
            pl.BlockSpec((8, 4 * R), lambda i, j: (i, j)),
        ],
        out_specs=pl.BlockSpec(memory_space=pltpu.SMEM),
        out_shape=jax.ShapeDtypeStruct((3,), jnp.float32),
        scratch_shapes=[pltpu.SMEM((3,), jnp.float32)],
    )(scores, assigned_labels, assigned_labels, assigned_labels,
      assigned_labels, off2, enc2)

    return {
        "total_loss": out[2],
        "regre_loss": out[1],
        "classification_loss": out[0],
    }


# speedup vs baseline: 1.1952x; 1.1952x over previous
import jax
import jax.numpy as jnp
from jax.experimental import pallas as pl
from jax.experimental.pallas import tpu as pltpu


def _body(s_ref, l_ref, o_ref, e_ref, out_ref, acc_ref):
    i = pl.program_id(0)
    j = pl.program_id(1)
    gi = pl.num_programs(0)
    gj = pl.num_programs(1)

    @pl.when((i == 0) & (j == 0))
    def _():
        acc_ref[0] = 0.0
        acc_ref[1] = 0.0
        acc_ref[2] = 0.0

    lab = jax.lax.bitcast_convert_type(l_ref[...], jnp.int32)
    labT = jnp.swapaxes(lab, 0, 1)
    fgT = (labT != 0).astype(jnp.float32)
    nfg_part = jnp.sum(fgT)

    d = o_ref[...] - e_ref[...]
    dT = jnp.swapaxes(d, 0, 1)
    R = lab.shape[1]
    dT3 = dT.reshape(R, 4, 8)
    ad = jnp.abs(dT3)
    sl1 = jnp.where(ad < 1.0, 0.5 * dT3 * dT3, ad - 0.5)
    sl1_part = jnp.sum(sl1 * fgT[:, None, :])

    C = s_ref.shape[2]
    s3 = s_ref[...].reshape(8 * R, C)
    ex = jnp.exp(s3)
    ones_v = jnp.ones((C, 128), dtype=jnp.float32)
    sum_ex = jax.lax.dot_general(
        ex, ones_v, (((1,), (0,)), ((), ())),
        preferred_element_type=jnp.float32,
    )[:, 0:1]
    lse_sum = jnp.sum(jnp.log(sum_ex))

    picked_sum = 0.0
    iota = jax.lax.broadcasted_iota(jnp.int32, (R, C), 1)
    for bb in range(8):
        s = s_ref[bb]
        lab_col = labT[:, bb : bb + 1]
        oh = jnp.where(iota == lab_col, s, 0.0)
        pick = jax.lax.dot_general(
            oh, ones_v, (((1,), (0,)), ((), ())),
            preferred_element_type=jnp.float32,
        )[:, 0:1]
        picked_sum += jnp.sum(pick)

    acc_ref[0] += lse_sum - picked_sum
    acc_ref[1] += nfg_part
    acc_ref[2] += sl1_part

    @pl.when((i == gi - 1) & (j == gj - 1))
    def _():
        nf = acc_ref[1]
        cls = acc_ref[0] / nf
        reg = acc_ref[2] / nf
        out_ref[0] = cls
        out_ref[1] = reg
        out_ref[2] = cls + reg


def kernel(offsets, scores, assigned_labels, encoded_bboxes):
    B, A, C = scores.shape
    R = 1024
    GB = B // 8
    GA = A // R

    lab2 = jax.lax.bitcast_convert_type(
        assigned_labels, jnp.float32
    ).reshape(B, A)
    off2 = offsets.reshape(B, A * 4)
    enc2 = encoded_bboxes.reshape(B, A * 4)

    out = pl.pallas_call(
        _body,
        grid=(GB, GA),
        in_specs=[
            pl.BlockSpec((8, R, C), lambda i, j: (i, j, 0)),
            pl.BlockSpec((8, R), lambda i, j: (i, j)),
            pl.BlockSpec((8, 4 * R), lambda i, j: (i, j)),
            pl.BlockSpec((8, 4 * R), lambda i, j: (i, j)),
        ],
        out_specs=pl.BlockSpec(memory_space=pltpu.SMEM),
        out_shape=jax.ShapeDtypeStruct((3,), jnp.float32),
        scratch_shapes=[pltpu.SMEM((3,), jnp.float32)],
    )(scores, lab2, off2, enc2)

    return {
        "total_loss": out[2],
        "regre_loss": out[1],
        "classification_loss": out[0],
    }


# speedup vs baseline: 1.2835x; 1.0739x over previous
import jax
import jax.numpy as jnp
from jax.experimental import pallas as pl
from jax.experimental.pallas import tpu as pltpu


def _body(s_ref, l_ref, o_ref, e_ref, out_ref, acc_ref):
    i = pl.program_id(0)
    j = pl.program_id(1)
    gi = pl.num_programs(0)
    gj = pl.num_programs(1)

    @pl.when((i == 0) & (j == 0))
    def _():
        acc_ref[0] = 0.0
        acc_ref[1] = 0.0
        acc_ref[2] = 0.0

    lab = jax.lax.bitcast_convert_type(l_ref[...], jnp.int32)
    labT = jnp.swapaxes(lab, 0, 1)
    fgT = (labT != 0).astype(jnp.float32)
    nfg_part = jnp.sum(fgT)

    d = o_ref[...] - e_ref[...]
    dT = jnp.swapaxes(d, 0, 1)
    R = lab.shape[1]
    dT3 = dT.reshape(R, 4, 8)
    ad = jnp.abs(dT3)
    sl1 = jnp.where(ad < 1.0, 0.5 * dT3 * dT3, ad - 0.5)
    sl1_part = jnp.sum(sl1 * fgT[:, None, :])

    C = s_ref.shape[2]
    s3 = s_ref[...].reshape(8 * R, C)
    ex = jnp.exp(s3)
    ones_v = jnp.ones((C, 128), dtype=jnp.float32)
    sum_ex = jax.lax.dot_general(
        ex, ones_v, (((1,), (0,)), ((), ())),
        preferred_element_type=jnp.float32,
    )[:, 0:1]
    lse_sum = jnp.sum(jnp.log(sum_ex))

    picked_sum = 0.0
    iota = jax.lax.broadcasted_iota(jnp.int32, (R, C), 1)
    for bb in range(8):
        s = s_ref[bb]
        lab_col = labT[:, bb : bb + 1]
        oh = jnp.where(iota == lab_col, s, 0.0)
        pick = jax.lax.dot_general(
            oh, ones_v, (((1,), (0,)), ((), ())),
            preferred_element_type=jnp.float32,
        )[:, 0:1]
        picked_sum += jnp.sum(pick)

    acc_ref[0] += lse_sum - picked_sum
    acc_ref[1] += nfg_part
    acc_ref[2] += sl1_part

    @pl.when((i == gi - 1) & (j == gj - 1))
    def _():
        nf = acc_ref[1]
        cls = acc_ref[0] / nf
        reg = acc_ref[2] / nf
        out_ref[0] = cls
        out_ref[1] = reg
        out_ref[2] = cls + reg


def kernel(offsets, scores, assigned_labels, encoded_bboxes):
    B, A, C = scores.shape
    R = 2048
    GB = B // 8
    GA = A // R

    lab2 = jax.lax.bitcast_convert_type(
        assigned_labels, jnp.float32
    ).reshape(B, A)
    off2 = offsets.reshape(B, A * 4)
    enc2 = encoded_bboxes.reshape(B, A * 4)

    out = pl.pallas_call(
        _body,
        grid=(GB, GA),
        in_specs=[
            pl.BlockSpec((8, R, C), lambda i, j: (i, j, 0)),
            pl.BlockSpec((8, R), lambda i, j: (i, j)),
            pl.BlockSpec((8, 4 * R), lambda i, j: (i, j)),
            pl.BlockSpec((8, 4 * R), lambda i, j: (i, j)),
        ],
        out_specs=pl.BlockSpec(memory_space=pltpu.SMEM),
        out_shape=jax.ShapeDtypeStruct((3,), jnp.float32),
        scratch_shapes=[pltpu.SMEM((3,), jnp.float32)],
    )(scores, lab2, off2, enc2)

    return {
        "total_loss": out[2],
        "regre_loss": out[1],
        "classification_loss": out[0],
    }


# speedup vs baseline: 1.3418x; 1.0454x over previous
import jax
import jax.numpy as jnp
from jax.experimental import pallas as pl
from jax.experimental.pallas import tpu as pltpu


def _body(s_ref, l_ref, o_ref, e_ref, out_ref, acc_ref):
    i = pl.program_id(0)
    j = pl.program_id(1)
    gi = pl.num_programs(0)
    gj = pl.num_programs(1)

    @pl.when((i == 0) & (j == 0))
    def _():
        acc_ref[0] = 0.0
        acc_ref[1] = 0.0
        acc_ref[2] = 0.0

    lab = jax.lax.bitcast_convert_type(l_ref[...], jnp.int32)
    labT = jnp.swapaxes(lab, 0, 1)
    fgT = (labT != 0).astype(jnp.float32)
    nfg_part = jnp.sum(fgT)

    d = o_ref[...] - e_ref[...]
    dT = jnp.swapaxes(d, 0, 1)
    R = lab.shape[1]
    dT3 = dT.reshape(R, 4, 8)
    ad = jnp.abs(dT3)
    sl1 = jnp.where(ad < 1.0, 0.5 * dT3 * dT3, ad - 0.5)
    sl1_part = jnp.sum(sl1 * fgT[:, None, :])

    C = s_ref.shape[2]
    s3 = s_ref[...].reshape(8 * R, C)
    ex = jnp.exp(s3)
    ones_v = jnp.ones((C, 128), dtype=jnp.float32)
    sum_ex = jax.lax.dot_general(
        ex, ones_v, (((1,), (0,)), ((), ())),
        preferred_element_type=jnp.float32,
    )[:, 0:1]
    lse_sum = jnp.sum(jnp.log(sum_ex))

    picked_sum = 0.0
    iota = jax.lax.broadcasted_iota(jnp.int32, (R, C), 1)
    for bb in range(8):
        s = s_ref[bb]
        lab_col = labT[:, bb : bb + 1]
        oh = jnp.where(iota == lab_col, s, 0.0)
        pick = jax.lax.dot_general(
            oh, ones_v, (((1,), (0,)), ((), ())),
            preferred_element_type=jnp.float32,
        )[:, 0:1]
        picked_sum += jnp.sum(pick)

    acc_ref[0] += lse_sum - picked_sum
    acc_ref[1] += nfg_part
    acc_ref[2] += sl1_part

    @pl.when((i == gi - 1) & (j == gj - 1))
    def _():
        nf = acc_ref[1]
        cls = acc_ref[0] / nf
        reg = acc_ref[2] / nf
        out_ref[0] = cls
        out_ref[1] = reg
        out_ref[2] = cls + reg


def kernel(offsets, scores, assigned_labels, encoded_bboxes):
    B, A, C = scores.shape
    R = 4096
    GB = B // 8
    GA = A // R

    lab2 = jax.lax.bitcast_convert_type(
        assigned_labels, jnp.float32
    ).reshape(B, A)
    off2 = offsets.reshape(B, A * 4)
    enc2 = encoded_bboxes.reshape(B, A * 4)

    out = pl.pallas_call(
        _body,
        grid=(GB, GA),
        in_specs=[
            pl.BlockSpec((8, R, C), lambda i, j: (i, j, 0)),
            pl.BlockSpec((8, R), lambda i, j: (i, j)),
            pl.BlockSpec((8, 4 * R), lambda i, j: (i, j)),
            pl.BlockSpec((8, 4 * R), lambda i, j: (i, j)),
        ],
        out_specs=pl.BlockSpec(memory_space=pltpu.SMEM),
        out_shape=jax.ShapeDtypeStruct((3,), jnp.float32),
        scratch_shapes=[pltpu.SMEM((3,), jnp.float32)],
    )(scores, lab2, off2, enc2)

    return {
        "total_loss": out[2],
        "regre_loss": out[1],
        "classification_loss": out[0],
    }
